# baseline (speedup 1.0000x reference)
import functools

import jax
import jax.numpy as jnp
from jax import lax
from jax.experimental import pallas as pl
from jax.experimental.pallas import tpu as pltpu
from jax.experimental.pallas import tpu_sc as plsc

NC = 2
NS = 16
LANES = 16
CH = 128
GB = 32
ZR = 128


def _round_up(v, m):
    return (v + m - 1) // m * m


def _mesh():
    return plsc.VectorSubcoreMesh(core_axis_name="c", subcore_axis_name="s")


def _zero_rows(zbuf, ncols):
    zr = zbuf.shape[0]

    @pl.loop(0, zr)
    def _(r):
        @pl.loop(0, ncols // LANES)
        def _(cc):
            zbuf[r, pl.ds(cc * LANES, LANES)] = jnp.zeros((LANES,), jnp.float32)


def _zero_acc_slice(zbuf, acc, row0, nrows):
    zr = zbuf.shape[0]

    @pl.loop(0, nrows // zr)
    def _(k):
        pltpu.sync_copy(zbuf, acc.at[pl.ds(row0 + k * zr, zr)])


def _sc_deg(dst2, n, nchp, nch):
    npad = _round_up(n + CH, NS * ZR)
    rpt = npad // NS
    cpt = nchp // (NC * NS)
    ngroups = cpt // GB

    @functools.partial(
        pl.kernel,
        out_type=jax.ShapeDtypeStruct((NC * npad, LANES), jnp.float32),
        mesh=_mesh(),
        scratch_types=[
            pltpu.VMEM((GB, CH), jnp.int32),
            pltpu.VMEM((CH, LANES), jnp.float32),
            pltpu.VMEM((ZR, LANES), jnp.float32),
            pltpu.VMEM_SHARED((npad, LANES), jnp.float32),
            pltpu.SemaphoreType.DMA,
        ],
    )
    def k(dst_hbm, out_hbm, dstB, ones_v, zbuf, acc, sem):
        c = lax.axis_index("c")
        s = lax.axis_index("s")

        @pl.loop(0, CH)
        def _(r):
            ones_v[r, :] = jnp.ones((LANES,), jnp.float32)

        _zero_rows(zbuf, LANES)
        _zero_acc_slice(zbuf, acc, s * rpt, rpt)
        plsc.subcore_barrier()

        first_c = (c * NS + s) * cpt

        @pl.loop(0, ngroups)
        def _(g):
            base = first_c + g * GB

            @pl.when(base < nch)
            def _():
                pltpu.sync_copy(dst_hbm.at[pl.ds(base, GB)], dstB)
                for b in range(GB):
                    @pl.when(base + b < nch)
                    def _(b=b):
                        pltpu.async_copy(ones_v, acc.at[dstB.at[b]], sem,
                                         add=True)
                for b in range(GB):
                    @pl.when(base + b < nch)
                    def _(b=b):
                        pltpu.make_async_copy(ones_v, acc.at[dstB.at[b]],
                                              sem).wait()

        plsc.subcore_barrier()
        pltpu.sync_copy(acc.at[pl.ds(s * rpt, rpt)],
                        out_hbm.at[pl.ds(c * npad + s * rpt, rpt)])

    return k(dst2).reshape(NC, npad, LANES)


def _sc_agg(table, src2, dst2, n, nchp, nch, feature_split):
    d = table.shape[1]
    npad = _round_up(n + CH, NS * ZR)
    rpt = npad // NS
    workers = NS if feature_split else NC * NS
    cpt = nchp // workers
    ngroups = cpt // GB

    @functools.partial(
        pl.kernel,
        out_type=jax.ShapeDtypeStruct((NC * npad, d), jnp.float32),
        mesh=_mesh(),
        scratch_types=[
            pltpu.VMEM((GB, CH), jnp.int32),
            pltpu.VMEM((GB, CH), jnp.int32),
            pltpu.VMEM((CH, d), jnp.float32),
            pltpu.VMEM((CH, d), jnp.float32),
            pltpu.VMEM_SHARED((npad, d), jnp.float32),
            pltpu.SemaphoreType.DMA,
            pltpu.SemaphoreType.DMA,
            pltpu.SemaphoreType.DMA,
            pltpu.SemaphoreType.DMA,
        ],
    )
    def k(tab_hbm, src_hbm, dst_hbm, out_hbm, srcB, dstB,
          rows0, rows1, acc, g0, g1, s0, s1):
        c = lax.axis_index("c")
        s = lax.axis_index("s")
        rows = (rows0, rows1)
        gsem = (g0, g1)
        ssem = (s0, s1)

        _zero_rows(rows0, d)
        _zero_acc_slice(rows0, acc, s * rpt, rpt)
        plsc.subcore_barrier()

        first_c = (s if feature_split else c * NS + s) * cpt

        def g_issue(base, b):
            @pl.when(base + b < nch)
            def _():
                pltpu.async_copy(tab_hbm.at[srcB.at[b]], rows[b % 2],
                                 gsem[b % 2])

        def g_wait_s_issue(base, b):
            @pl.when(base + b < nch)
            def _():
                pltpu.make_async_copy(tab_hbm.at[srcB.at[b]], rows[b % 2],
                                      gsem[b % 2]).wait()
                pltpu.async_copy(rows[b % 2], acc.at[dstB.at[b]],
                                 ssem[b % 2], add=True)

        def s_wait(base, b):
            @pl.when(base + b < nch)
            def _():
                pltpu.make_async_copy(rows[b % 2], acc.at[dstB.at[b]],
                                      ssem[b % 2]).wait()

        @pl.loop(0, ngroups)
        def _(g):
            base = first_c + g * GB

            @pl.when(base < nch)
            def _():
                pltpu.sync_copy(src_hbm.at[pl.ds(base, GB)], srcB)
                pltpu.sync_copy(dst_hbm.at[pl.ds(base, GB)], dstB)
                if feature_split:
                    @pl.loop(0, GB)
                    def _(bb):
                        @pl.loop(0, CH // LANES)
                        def _(kk):
                            sl = pl.ds(kk * LANES, LANES)
                            srcB[bb, sl] = srcB[bb, sl] + c * n
                for b in range(GB):
                    if b >= 2:
                        s_wait(base, b - 2)
                    g_issue(base, b)
                    if b >= 1:
                        g_wait_s_issue(base, b - 1)
                g_wait_s_issue(base, GB - 1)
                s_wait(base, GB - 2)
                s_wait(base, GB - 1)

        plsc.subcore_barrier()
        pltpu.sync_copy(acc.at[pl.ds(s * rpt, rpt)],
                        out_hbm.at[pl.ds(c * npad + s * rpt, rpt)])

    return k(table, src2, dst2).reshape(NC, npad, d)


def _tc_z1(x, w, n, blk):
    grid = (n // blk,)

    def body(x_ref, w_ref, o_ref):
        o_ref[...] = jnp.dot(x_ref[...], w_ref[...],
                             preferred_element_type=jnp.float32)

    return pl.pallas_call(
        body,
        grid=grid,
        in_specs=[
            pl.BlockSpec((blk, x.shape[1]), lambda i: (i, 0)),
            pl.BlockSpec(w.shape, lambda i: (0, 0)),
        ],
        out_specs=pl.BlockSpec((blk, w.shape[1]), lambda i: (i, 0)),
        out_shape=jax.ShapeDtypeStruct((n, w.shape[1]), jnp.float32),
    )(x, w)


def _tc_scale_split(z, hist, n, blk):
    dh = z.shape[1]
    hd = dh // 2
    grid = (n // blk,)

    def body(z_ref, h_ref, dv_ref, o_ref):
        deg = h_ref[0, :, 0:1] + h_ref[1, :, 0:1] + 1.0
        dv = lax.rsqrt(deg)
        dv_ref[...] = dv
        zz = z_ref[...] * dv
        o_ref[0] = zz[:, :hd]
        o_ref[1] = zz[:, hd:]

    return pl.pallas_call(
        body,
        grid=grid,
        in_specs=[
            pl.BlockSpec((blk, dh), lambda i: (i, 0)),
            pl.BlockSpec((2, blk, LANES), lambda i: (0, i, 0)),
        ],
        out_specs=[
            pl.BlockSpec((blk, 1), lambda i: (i, 0)),
            pl.BlockSpec((2, blk, hd), lambda i: (0, i, 0)),
        ],
        out_shape=[
            jax.ShapeDtypeStruct((n, 1), jnp.float32),
            jax.ShapeDtypeStruct((2, n, hd), jnp.float32),
        ],
    )(z, hist)


def _tc_mid(agg1, hsp1, dinv, b1, w2, n, blk):
    grid = (n // blk,)
    hd = agg1.shape[2]

    def body(a_ref, h_ref, dv_ref, b_ref, w_ref, h1_ref, h2p_ref):
        lo = a_ref[0] + h_ref[0]
        hi = a_ref[1] + h_ref[1]
        pre = jnp.concatenate([lo, hi], axis=1)
        h1 = jax.nn.relu(pre * dv_ref[...] + b_ref[...])
        h1_ref[...] = h1
        z2 = jnp.dot(h1, w_ref[...], preferred_element_type=jnp.float32)
        h2p_ref[...] = z2 * dv_ref[...]

    return pl.pallas_call(
        body,
        grid=grid,
        in_specs=[
            pl.BlockSpec((2, blk, hd), lambda i: (0, i, 0)),
            pl.BlockSpec((2, blk, hd), lambda i: (0, i, 0)),
            pl.BlockSpec((blk, 1), lambda i: (i, 0)),
            pl.BlockSpec((1, 2 * hd), lambda i: (0, 0)),
            pl.BlockSpec(w2.shape, lambda i: (0, 0)),
        ],
        out_specs=[
            pl.BlockSpec((blk, 2 * hd), lambda i: (i, 0)),
            pl.BlockSpec((blk, w2.shape[1]), lambda i: (i, 0)),
        ],
        out_shape=[
            jax.ShapeDtypeStruct((n, 2 * hd), jnp.float32),
            jax.ShapeDtypeStruct((n, w2.shape[1]), jnp.float32),
        ],
    )(agg1, hsp1, dinv, b1, w2)


def _tc_out(parts, h2p, dinv, b2, n, blk):
    grid = (n // blk,)
    do = h2p.shape[1]

    def body(p_ref, h_ref, dv_ref, b_ref, o_ref):
        agg = p_ref[0] + p_ref[1] + h_ref[...]
        o_ref[...] = agg * dv_ref[...] + b_ref[...]

    return pl.pallas_call(
        body,
        grid=grid,
        in_specs=[
            pl.BlockSpec((2, blk, do), lambda i: (0, i, 0)),
            pl.BlockSpec((blk, do), lambda i: (i, 0)),
            pl.BlockSpec((blk, 1), lambda i: (i, 0)),
            pl.BlockSpec((1, do), lambda i: (0, 0)),
        ],
        out_specs=pl.BlockSpec((blk, do), lambda i: (i, 0)),
        out_shape=jax.ShapeDtypeStruct((n, do), jnp.float32),
    )(parts, h2p, dinv, b2)


def kernel(x, edge_index, W1, b1, W2, b2):
    n = x.shape[0]
    e = edge_index.shape[1]
    blk = 1000

    nch = -(-e // CH)
    cpt_fs = _round_up(-(-nch // NS), GB)
    cpt_es = _round_up(-(-nch // (NC * NS)), GB)
    nchp = max(NS * cpt_fs, NC * NS * cpt_es)
    pad = nchp * CH - e
    src2 = jnp.pad(edge_index[0], (0, pad)).reshape(nchp, CH)
    dst_fill = n + jnp.arange(pad, dtype=edge_index.dtype) % CH
    dst2 = jnp.concatenate([edge_index[1], dst_fill]).reshape(nchp, CH)

    hist = _sc_deg(dst2, n, nchp, nch)
    z1 = _tc_z1(x, W1, n, blk)
    dinv, hsp1 = _tc_scale_split(z1, hist, n, blk)
    agg1 = _sc_agg(hsp1.reshape(NC * n, -1), src2, dst2, n, nchp, nch, True)
    h1, h2p = _tc_mid(agg1, hsp1, dinv, b1.reshape(1, -1), W2, n, blk)
    parts = _sc_agg(h2p, src2, dst2, n, nchp, nch, False)
    h2 = _tc_out(parts, h2p, dinv, b2.reshape(1, -1), n, blk)
    return (h2, h1)


# speedup vs baseline: 1.0688x; 1.0688x over previous
import functools

import jax
import jax.numpy as jnp
from jax import lax
from jax.experimental import pallas as pl
from jax.experimental.pallas import tpu as pltpu
from jax.experimental.pallas import tpu_sc as plsc

NC = 2
NS = 16
LANES = 16
CH = 128
GB = 16
ZR = 128


def _round_up(v, m):
    return (v + m - 1) // m * m


def _mesh():
    return plsc.VectorSubcoreMesh(core_axis_name="c", subcore_axis_name="s")


def _zero_rows(zbuf, ncols):
    zr = zbuf.shape[0]

    @pl.loop(0, zr)
    def _(r):
        @pl.loop(0, ncols // LANES)
        def _(cc):
            zbuf[r, pl.ds(cc * LANES, LANES)] = jnp.zeros((LANES,), jnp.float32)


def _zero_acc_slice(zbuf, acc, row0, nrows):
    zr = zbuf.shape[0]

    @pl.loop(0, nrows // zr)
    def _(k):
        pltpu.sync_copy(zbuf, acc.at[pl.ds(row0 + k * zr, zr)])


def _sc_deg(dst2, n, nchp, nch):
    npad = _round_up(n + CH, NS * ZR)
    rpt = npad // NS
    cpt = nchp // (NC * NS)
    ngroups = cpt // GB

    @functools.partial(
        pl.kernel,
        out_type=jax.ShapeDtypeStruct((NC * npad, LANES), jnp.float32),
        mesh=_mesh(),
        scratch_types=[
            pltpu.VMEM((GB, CH), jnp.int32),
            pltpu.VMEM((CH, LANES), jnp.float32),
            pltpu.VMEM((ZR, LANES), jnp.float32),
            pltpu.VMEM_SHARED((npad, LANES), jnp.float32),
            pltpu.SemaphoreType.DMA,
        ],
    )
    def k(dst_hbm, out_hbm, dstB, ones_v, zbuf, acc, sem):
        c = lax.axis_index("c")
        s = lax.axis_index("s")

        @pl.loop(0, CH)
        def _(r):
            ones_v[r, :] = jnp.ones((LANES,), jnp.float32)

        _zero_rows(zbuf, LANES)
        _zero_acc_slice(zbuf, acc, s * rpt, rpt)
        plsc.subcore_barrier()

        first_c = (c * NS + s) * cpt

        @pl.loop(0, ngroups)
        def _(g):
            base = first_c + g * GB

            @pl.when(base < nch)
            def _():
                pltpu.sync_copy(dst_hbm.at[pl.ds(base, GB)], dstB)
                for b in range(GB):
                    @pl.when(base + b < nch)
                    def _(b=b):
                        pltpu.async_copy(ones_v, acc.at[dstB.at[b]], sem,
                                         add=True)
                for b in range(GB):
                    @pl.when(base + b < nch)
                    def _(b=b):
                        pltpu.make_async_copy(ones_v, acc.at[dstB.at[b]],
                                              sem).wait()

        plsc.subcore_barrier()
        pltpu.sync_copy(acc.at[pl.ds(s * rpt, rpt)],
                        out_hbm.at[pl.ds(c * npad + s * rpt, rpt)])

    return k(dst2).reshape(NC, npad, LANES)


def _sc_agg(table, src2, dst2, n, nchp, nch, feature_split):
    d = table.shape[1]
    npad = _round_up(n + CH, NS * ZR)
    rpt = npad // NS
    workers = NS if feature_split else NC * NS
    cpt = nchp // workers
    ngroups = cpt // GB

    @functools.partial(
        pl.kernel,
        out_type=jax.ShapeDtypeStruct((NC * npad, d), jnp.float32),
        mesh=_mesh(),
        scratch_types=[
            pltpu.VMEM((GB, CH), jnp.int32),
            pltpu.VMEM((GB, CH), jnp.int32),
            pltpu.VMEM((CH, d), jnp.float32),
            pltpu.VMEM((CH, d), jnp.float32),
            pltpu.VMEM_SHARED((npad, d), jnp.float32),
            pltpu.SemaphoreType.DMA,
            pltpu.SemaphoreType.DMA,
            pltpu.SemaphoreType.DMA,
            pltpu.SemaphoreType.DMA,
        ],
    )
    def k(tab_hbm, src_hbm, dst_hbm, out_hbm, srcB, dstB,
          rows0, rows1, acc, g0, g1, s0, s1):
        c = lax.axis_index("c")
        s = lax.axis_index("s")
        rows = (rows0, rows1)
        gsem = (g0, g1)
        ssem = (s0, s1)

        _zero_rows(rows0, d)
        _zero_acc_slice(rows0, acc, s * rpt, rpt)
        plsc.subcore_barrier()

        first_c = (s if feature_split else c * NS + s) * cpt

        def g_issue(base, b):
            @pl.when(base + b < nch)
            def _():
                pltpu.async_copy(tab_hbm.at[srcB.at[b]], rows[b % 2],
                                 gsem[b % 2])

        def g_wait_s_issue(base, b):
            @pl.when(base + b < nch)
            def _():
                pltpu.make_async_copy(tab_hbm.at[srcB.at[b]], rows[b % 2],
                                      gsem[b % 2]).wait()
                pltpu.async_copy(rows[b % 2], acc.at[dstB.at[b]],
                                 ssem[b % 2], add=True)

        def s_wait(base, b):
            @pl.when(base + b < nch)
            def _():
                pltpu.make_async_copy(rows[b % 2], acc.at[dstB.at[b]],
                                      ssem[b % 2]).wait()

        @pl.loop(0, ngroups)
        def _(g):
            base = first_c + g * GB

            @pl.when(base < nch)
            def _():
                pltpu.sync_copy(src_hbm.at[pl.ds(base, GB)], srcB)
                pltpu.sync_copy(dst_hbm.at[pl.ds(base, GB)], dstB)
                if feature_split:
                    @pl.loop(0, GB)
                    def _(bb):
                        @pl.loop(0, CH // LANES)
                        def _(kk):
                            sl = pl.ds(kk * LANES, LANES)
                            srcB[bb, sl] = srcB[bb, sl] + c * n
                for b in range(GB):
                    if b >= 2:
                        s_wait(base, b - 2)
                    g_issue(base, b)
                    if b >= 1:
                        g_wait_s_issue(base, b - 1)
                g_wait_s_issue(base, GB - 1)
                s_wait(base, GB - 2)
                s_wait(base, GB - 1)

        plsc.subcore_barrier()
        pltpu.sync_copy(acc.at[pl.ds(s * rpt, rpt)],
                        out_hbm.at[pl.ds(c * npad + s * rpt, rpt)])

    return k(table, src2, dst2).reshape(NC, npad, d)


def _tc_z1(x, w, n, blk):
    grid = (n // blk,)

    def body(x_ref, w_ref, o_ref):
        o_ref[...] = jnp.dot(x_ref[...], w_ref[...],
                             preferred_element_type=jnp.float32)

    return pl.pallas_call(
        body,
        grid=grid,
        in_specs=[
            pl.BlockSpec((blk, x.shape[1]), lambda i: (i, 0)),
            pl.BlockSpec(w.shape, lambda i: (0, 0)),
        ],
        out_specs=pl.BlockSpec((blk, w.shape[1]), lambda i: (i, 0)),
        out_shape=jax.ShapeDtypeStruct((n, w.shape[1]), jnp.float32),
    )(x, w)


def _tc_scale_split(z, hist, n, blk):
    dh = z.shape[1]
    hd = dh // 2
    grid = (n // blk,)

    def body(z_ref, h_ref, dv_ref, o_ref):
        deg = h_ref[0, :, 0:1] + h_ref[1, :, 0:1] + 1.0
        dv = lax.rsqrt(deg)
        dv_ref[...] = dv
        zz = z_ref[...] * dv
        o_ref[0] = zz[:, :hd]
        o_ref[1] = zz[:, hd:]

    return pl.pallas_call(
        body,
        grid=grid,
        in_specs=[
            pl.BlockSpec((blk, dh), lambda i: (i, 0)),
            pl.BlockSpec((2, blk, LANES), lambda i: (0, i, 0)),
        ],
        out_specs=[
            pl.BlockSpec((blk, 1), lambda i: (i, 0)),
            pl.BlockSpec((2, blk, hd), lambda i: (0, i, 0)),
        ],
        out_shape=[
            jax.ShapeDtypeStruct((n, 1), jnp.float32),
            jax.ShapeDtypeStruct((2, n, hd), jnp.float32),
        ],
    )(z, hist)


def _tc_mid(agg1, hsp1, dinv, b1, w2, n, blk):
    grid = (n // blk,)
    hd = agg1.shape[2]

    def body(a_ref, h_ref, dv_ref, b_ref, w_ref, h1_ref, h2p_ref):
        lo = a_ref[0] + h_ref[0]
        hi = a_ref[1] + h_ref[1]
        pre = jnp.concatenate([lo, hi], axis=1)
        h1 = jax.nn.relu(pre * dv_ref[...] + b_ref[...])
        h1_ref[...] = h1
        z2 = jnp.dot(h1, w_ref[...], preferred_element_type=jnp.float32)
        h2p_ref[...] = z2 * dv_ref[...]

    return pl.pallas_call(
        body,
        grid=grid,
        in_specs=[
            pl.BlockSpec((2, blk, hd), lambda i: (0, i, 0)),
            pl.BlockSpec((2, blk, hd), lambda i: (0, i, 0)),
            pl.BlockSpec((blk, 1), lambda i: (i, 0)),
            pl.BlockSpec((1, 2 * hd), lambda i: (0, 0)),
            pl.BlockSpec(w2.shape, lambda i: (0, 0)),
        ],
        out_specs=[
            pl.BlockSpec((blk, 2 * hd), lambda i: (i, 0)),
            pl.BlockSpec((blk, w2.shape[1]), lambda i: (i, 0)),
        ],
        out_shape=[
            jax.ShapeDtypeStruct((n, 2 * hd), jnp.float32),
            jax.ShapeDtypeStruct((n, w2.shape[1]), jnp.float32),
        ],
    )(agg1, hsp1, dinv, b1, w2)


def _tc_out(parts, h2p, dinv, b2, n, blk):
    grid = (n // blk,)
    do = h2p.shape[1]

    def body(p_ref, h_ref, dv_ref, b_ref, o_ref):
        agg = p_ref[0] + p_ref[1] + h_ref[...]
        o_ref[...] = agg * dv_ref[...] + b_ref[...]

    return pl.pallas_call(
        body,
        grid=grid,
        in_specs=[
            pl.BlockSpec((2, blk, do), lambda i: (0, i, 0)),
            pl.BlockSpec((blk, do), lambda i: (i, 0)),
            pl.BlockSpec((blk, 1), lambda i: (i, 0)),
            pl.BlockSpec((1, do), lambda i: (0, 0)),
        ],
        out_specs=pl.BlockSpec((blk, do), lambda i: (i, 0)),
        out_shape=jax.ShapeDtypeStruct((n, do), jnp.float32),
    )(parts, h2p, dinv, b2)


def kernel(x, edge_index, W1, b1, W2, b2):
    n = x.shape[0]
    e = edge_index.shape[1]
    blk = 1000

    nch = -(-e // CH)
    cpt_fs = _round_up(-(-nch // NS), GB)
    cpt_es = _round_up(-(-nch // (NC * NS)), GB)
    nchp = max(NS * cpt_fs, NC * NS * cpt_es)
    pad = nchp * CH - e
    src2 = jnp.pad(edge_index[0], (0, pad)).reshape(nchp, CH)
    dst2 = jnp.pad(edge_index[1], (0, pad)).reshape(nchp, CH)

    hist = _sc_deg(dst2, n, nchp, nch)
    z1 = _tc_z1(x, W1, n, blk)
    dinv, hsp1 = _tc_scale_split(z1, hist, n, blk)
    agg1 = _sc_agg(hsp1.reshape(NC * n, -1), src2, dst2, n, nchp, nch, True)
    h1, h2p = _tc_mid(agg1, hsp1, dinv, b1.reshape(1, -1), W2, n, blk)
    parts = _sc_agg(h2p, src2, dst2, n, nchp, nch, False)
    h2 = _tc_out(parts, h2p, dinv, b2.reshape(1, -1), n, blk)
    return (h2, h1)


# speedup vs baseline: 1.0845x; 1.0147x over previous
import dataclasses
import functools

import jax
import jax.numpy as jnp
from jax import lax
from jax.experimental import pallas as pl
from jax.experimental.pallas import tpu as pltpu
from jax.experimental.pallas import tpu_sc as plsc

NC = 2
NS = 16
LANES = 16
CH = 128
GB = 16
ZR = 128


def _round_up(v, m):
    return (v + m - 1) // m * m


def _mesh():
    return plsc.VectorSubcoreMesh(core_axis_name="c", subcore_axis_name="s")


def _zero_rows(zbuf, ncols):
    zr = zbuf.shape[0]

    @pl.loop(0, zr)
    def _(r):
        @pl.loop(0, ncols // LANES)
        def _(cc):
            zbuf[r, pl.ds(cc * LANES, LANES)] = jnp.zeros((LANES,), jnp.float32)


def _zero_acc_slice(zbuf, acc, row0, nrows):
    zr = zbuf.shape[0]

    @pl.loop(0, nrows // zr)
    def _(k):
        pltpu.sync_copy(zbuf, acc.at[pl.ds(row0 + k * zr, zr)])


def _sc_deg(dst2, n, nchp, nch):
    npad = _round_up(n, 128)
    cpt = nchp // (NC * NS)
    ngroups = cpt // GB

    @functools.partial(
        pl.kernel,
        out_type=jax.ShapeDtypeStruct((NC * NS, npad), jnp.float32),
        mesh=_mesh(),
        scratch_types=[
            pltpu.VMEM((GB, CH), jnp.int32),
            pltpu.VMEM((npad,), jnp.float32),
        ],
        compiler_params=dataclasses.replace(
            pltpu.CompilerParams(), needs_layout_passes=False),
    )
    def k(dst_hbm, out_hbm, dstB, acc):
        c = lax.axis_index("c")
        s = lax.axis_index("s")
        wid = c * NS + s

        @pl.loop(0, npad // LANES)
        def _(r):
            acc[pl.ds(r * LANES, LANES)] = jnp.zeros((LANES,), jnp.float32)

        first_c = wid * cpt
        ones = jnp.ones((LANES,), jnp.float32)

        @pl.loop(0, ngroups)
        def _(g):
            base = first_c + g * GB

            @pl.when(base < nch)
            def _():
                pltpu.sync_copy(dst_hbm.at[pl.ds(base, GB)], dstB)
                for b in range(GB):
                    @pl.when(base + b < nch)
                    def _(b=b):
                        @pl.loop(0, CH // LANES)
                        def _(k2):
                            iv = dstB[b, pl.ds(k2 * LANES, LANES)]
                            plsc.addupdate_scatter(acc, [iv], ones)

        pltpu.sync_copy(acc, out_hbm.at[wid])

    return k(dst2)


def _sc_agg(table, srca, srcb, dst2, n, nchp, nch, feature_split):
    d = table.shape[1]
    npad = _round_up(n + CH, NS * ZR)
    rpt = npad // NS
    workers = NS if feature_split else NC * NS
    cpt = nchp // workers
    ngroups = cpt // GB

    @functools.partial(
        pl.kernel,
        out_type=jax.ShapeDtypeStruct((NC * npad, d), jnp.float32),
        mesh=_mesh(),
        scratch_types=[
            pltpu.VMEM((GB, CH), jnp.int32),
            pltpu.VMEM((GB, CH), jnp.int32),
            pltpu.VMEM((CH, d), jnp.float32),
            pltpu.VMEM((CH, d), jnp.float32),
            pltpu.VMEM_SHARED((npad, d), jnp.float32),
            pltpu.SemaphoreType.DMA,
            pltpu.SemaphoreType.DMA,
            pltpu.SemaphoreType.DMA,
            pltpu.SemaphoreType.DMA,
        ],
        compiler_params=dataclasses.replace(
            pltpu.CompilerParams(), needs_layout_passes=False),
    )
    def k(tab_hbm, srca_hbm, srcb_hbm, dst_hbm, out_hbm, srcB, dstB,
          rows0, rows1, acc, g0, g1, s0, s1):
        c = lax.axis_index("c")
        s = lax.axis_index("s")
        rows = (rows0, rows1)
        gsem = (g0, g1)
        ssem = (s0, s1)

        _zero_rows(rows0, d)
        _zero_acc_slice(rows0, acc, s * rpt, rpt)
        plsc.subcore_barrier()

        first_c = (s if feature_split else c * NS + s) * cpt

        def g_issue(base, b):
            @pl.when(base + b < nch)
            def _():
                pltpu.async_copy(tab_hbm.at[srcB.at[b]], rows[b % 2],
                                 gsem[b % 2])

        def g_wait_s_issue(base, b):
            @pl.when(base + b < nch)
            def _():
                pltpu.make_async_copy(tab_hbm.at[srcB.at[b]], rows[b % 2],
                                      gsem[b % 2]).wait()
                pltpu.async_copy(rows[b % 2], acc.at[dstB.at[b]],
                                 ssem[b % 2], add=True)

        def s_wait(base, b):
            @pl.when(base + b < nch)
            def _():
                pltpu.make_async_copy(rows[b % 2], acc.at[dstB.at[b]],
                                      ssem[b % 2]).wait()

        @pl.loop(0, ngroups)
        def _(g):
            base = first_c + g * GB

            @pl.when(base < nch)
            def _():
                @pl.when(c == 0)
                def _():
                    pltpu.sync_copy(srca_hbm.at[pl.ds(base, GB)], srcB)

                @pl.when(c == 1)
                def _():
                    pltpu.sync_copy(srcb_hbm.at[pl.ds(base, GB)], srcB)

                pltpu.sync_copy(dst_hbm.at[pl.ds(base, GB)], dstB)
                for b in range(GB):
                    if b >= 2:
                        s_wait(base, b - 2)
                    g_issue(base, b)
                    if b >= 1:
                        g_wait_s_issue(base, b - 1)
                g_wait_s_issue(base, GB - 1)
                s_wait(base, GB - 2)
                s_wait(base, GB - 1)

        plsc.subcore_barrier()
        pltpu.sync_copy(acc.at[pl.ds(s * rpt, rpt)],
                        out_hbm.at[pl.ds(c * npad + s * rpt, rpt)])

    return k(table, srca, srcb, dst2).reshape(NC, npad, d)


def _tc_z1(x, w, n, blk):
    grid = (n // blk,)

    def body(x_ref, w_ref, o_ref):
        o_ref[...] = jnp.dot(x_ref[...], w_ref[...],
                             preferred_element_type=jnp.float32)

    return pl.pallas_call(
        body,
        grid=grid,
        in_specs=[
            pl.BlockSpec((blk, x.shape[1]), lambda i: (i, 0)),
            pl.BlockSpec(w.shape, lambda i: (0, 0)),
        ],
        out_specs=pl.BlockSpec((blk, w.shape[1]), lambda i: (i, 0)),
        out_shape=jax.ShapeDtypeStruct((n, w.shape[1]), jnp.float32),
    )(x, w)


def _tc_dinv(hist, n):
    def body(h_ref, o_ref):
        deg = jnp.sum(h_ref[...], axis=0, keepdims=True) + 1.0
        dv = lax.transpose(lax.rsqrt(deg), (1, 0))
        o_ref[...] = dv[:n, :]

    return pl.pallas_call(
        body,
        out_shape=jax.ShapeDtypeStruct((n, 1), jnp.float32),
    )(hist)


def _tc_scale_split(z, dinv, n, blk):
    dh = z.shape[1]
    hd = dh // 2
    grid = (n // blk,)

    def body(z_ref, dv_ref, o_ref):
        zz = z_ref[...] * dv_ref[...]
        o_ref[0] = zz[:, :hd]
        o_ref[1] = zz[:, hd:]

    return pl.pallas_call(
        body,
        grid=grid,
        in_specs=[
            pl.BlockSpec((blk, dh), lambda i: (i, 0)),
            pl.BlockSpec((blk, 1), lambda i: (i, 0)),
        ],
        out_specs=pl.BlockSpec((2, blk, hd), lambda i: (0, i, 0)),
        out_shape=jax.ShapeDtypeStruct((2, n, hd), jnp.float32),
    )(z, dinv)


def _tc_mid(agg1, hsp1, dinv, b1, w2, n, blk):
    grid = (n // blk,)
    hd = agg1.shape[2]

    def body(a_ref, h_ref, dv_ref, b_ref, w_ref, h1_ref, h2p_ref):
        lo = a_ref[0] + h_ref[0]
        hi = a_ref[1] + h_ref[1]
        pre = jnp.concatenate([lo, hi], axis=1)
        h1 = jax.nn.relu(pre * dv_ref[...] + b_ref[...])
        h1_ref[...] = h1
        z2 = jnp.dot(h1, w_ref[...], preferred_element_type=jnp.float32)
        h2p_ref[...] = z2 * dv_ref[...]

    return pl.pallas_call(
        body,
        grid=grid,
        in_specs=[
            pl.BlockSpec((2, blk, hd), lambda i: (0, i, 0)),
            pl.BlockSpec((2, blk, hd), lambda i: (0, i, 0)),
            pl.BlockSpec((blk, 1), lambda i: (i, 0)),
            pl.BlockSpec((1, 2 * hd), lambda i: (0, 0)),
            pl.BlockSpec(w2.shape, lambda i: (0, 0)),
        ],
        out_specs=[
            pl.BlockSpec((blk, 2 * hd), lambda i: (i, 0)),
            pl.BlockSpec((blk, w2.shape[1]), lambda i: (i, 0)),
        ],
        out_shape=[
            jax.ShapeDtypeStruct((n, 2 * hd), jnp.float32),
            jax.ShapeDtypeStruct((n, w2.shape[1]), jnp.float32),
        ],
    )(agg1, hsp1, dinv, b1, w2)


def _tc_out(parts, h2p, dinv, b2, n, blk):
    grid = (n // blk,)
    do = h2p.shape[1]

    def body(p_ref, h_ref, dv_ref, b_ref, o_ref):
        agg = p_ref[0] + p_ref[1] + h_ref[...]
        o_ref[...] = agg * dv_ref[...] + b_ref[...]

    return pl.pallas_call(
        body,
        grid=grid,
        in_specs=[
            pl.BlockSpec((2, blk, do), lambda i: (0, i, 0)),
            pl.BlockSpec((blk, do), lambda i: (i, 0)),
            pl.BlockSpec((blk, 1), lambda i: (i, 0)),
            pl.BlockSpec((1, do), lambda i: (0, 0)),
        ],
        out_specs=pl.BlockSpec((blk, do), lambda i: (i, 0)),
        out_shape=jax.ShapeDtypeStruct((n, do), jnp.float32),
    )(parts, h2p, dinv, b2)


def kernel(x, edge_index, W1, b1, W2, b2):
    n = x.shape[0]
    e = edge_index.shape[1]
    blk = 1000

    nch = -(-e // CH)
    cpt_fs = _round_up(-(-nch // NS), GB)
    cpt_es = _round_up(-(-nch // (NC * NS)), GB)
    nchp = max(NS * cpt_fs, NC * NS * cpt_es)
    pad = nchp * CH - e
    src2 = jnp.pad(edge_index[0], (0, pad)).reshape(nchp, CH)
    dst2 = jnp.pad(edge_index[1], (0, pad)).reshape(nchp, CH)

    hist = _sc_deg(dst2, n, nchp, nch)
    z1 = _tc_z1(x, W1, n, blk)
    dinv = _tc_dinv(hist, n)
    hsp1 = _tc_scale_split(z1, dinv, n, blk)
    src2n = src2 + n
    agg1 = _sc_agg(hsp1.reshape(NC * n, -1), src2, src2n, dst2,
                   n, nchp, nch, True)
    h1, h2p = _tc_mid(agg1, hsp1, dinv, b1.reshape(1, -1), W2, n, blk)
    parts = _sc_agg(h2p, src2, src2, dst2, n, nchp, nch, False)
    h2 = _tc_out(parts, h2p, dinv, b2.reshape(1, -1), n, blk)
    return (h2, h1)
